# baseline (speedup 1.0000x reference)
import functools

import jax
import jax.numpy as jnp
from jax import lax
from jax.experimental import pallas as pl
from jax.experimental.pallas import tpu as pltpu
from jax.experimental.pallas import tpu_sc as plsc

_NC = 2
_NS = 16
_NW = _NC * _NS
_IPG = 128
_CHUNK = 512
_PBLK = 256
_BBLK = 512


@functools.lru_cache(maxsize=None)
def _make_gather(n_rows: int, table_rows: int, d: int):
    assert n_rows % (_NW * _CHUNK) == 0
    b_per_w = n_rows // _NW
    n_chunks = b_per_w // _CHUNK
    n_sub = _CHUNK // _IPG
    idx_rows_per_w = b_per_w // _IPG
    mesh = plsc.VectorSubcoreMesh(core_axis_name="c", subcore_axis_name="s")

    @functools.partial(
        pl.kernel,
        mesh=mesh,
        out_type=jax.ShapeDtypeStruct((n_rows, d), jnp.float32),
        scratch_types=[
            pltpu.VMEM((n_sub, _IPG), jnp.int32),
            pltpu.VMEM((_CHUNK, d), jnp.float32),
            pltpu.SemaphoreType.DMA,
        ],
        compiler_params=pltpu.CompilerParams(use_tc_tiling_on_sc=False),
    )
    def gather_kernel(table_hbm, idx_hbm, out_hbm, idx_v, rows_v, sem):
        wid = lax.axis_index("s") * _NC + lax.axis_index("c")
        row_base = wid * b_per_w
        idx_row_base = wid * idx_rows_per_w

        def body(g, carry):
            pltpu.sync_copy(idx_hbm.at[pl.ds(idx_row_base + g * n_sub, n_sub)],
                            idx_v)
            copies = [
                pltpu.async_copy(table_hbm.at[idx_v.at[j]],
                                 rows_v.at[pl.ds(j * _IPG, _IPG)], sem)
                for j in range(n_sub)
            ]
            for c in copies:
                c.wait()
            pltpu.sync_copy(rows_v, out_hbm.at[pl.ds(row_base + g * _CHUNK,
                                                     _CHUNK)])
            return carry

        lax.fori_loop(0, n_chunks, body, 0)

    return gather_kernel


def _combine_body(n_valid, n_pairs, x1_ref, x2_ref, sp_ref, o_ref):
    i = pl.program_id(0)
    ta = x1_ref[...].T
    tb = x2_ref[...].T
    rows_b = (lax.broadcasted_iota(jnp.int32, (_PBLK, 1), 0)
              + (n_pairs + i * _PBLK))
    s_idx = jnp.clip(rows_b - n_valid, 0, 4)
    for k in range(1, 5):
        tb = tb + jnp.where(s_idx == k, 1.0, 0.0) * sp_ref[k, :][None, :]
    o_ref[:, 0:64] = ta
    o_ref[:, 64:128] = tb


@functools.lru_cache(maxsize=None)
def _make_combine(vocab: int, d: int, n_valid: int, n_pairs: int):
    n_blk = n_pairs // _PBLK
    return pl.pallas_call(
        functools.partial(_combine_body, n_valid, n_pairs),
        grid=(n_blk,),
        in_specs=[
            pl.BlockSpec((d, _PBLK), lambda i: (0, i)),
            pl.BlockSpec((d, _PBLK), lambda i: (0, n_blk + i)),
            pl.BlockSpec((8, d), lambda i: (0, 0)),
        ],
        out_specs=pl.BlockSpec((_PBLK, 2 * d), lambda i: (i, 0)),
        out_shape=jax.ShapeDtypeStruct((n_pairs, 2 * d), jnp.float32),
    )


def _tpose_body(x_ref, o_ref):
    x = x_ref[...]
    o_ref[0, :, 0:_BBLK // 2] = x[:, 0:64].T
    o_ref[0, :, _BBLK // 2:_BBLK] = x[:, 64:128].T


@functools.lru_cache(maxsize=None)
def _make_tpose(seq: int, batch: int, d: int):
    n_b = batch // _BBLK
    return pl.pallas_call(
        _tpose_body,
        grid=(seq, n_b),
        in_specs=[pl.BlockSpec((_BBLK // 2, 128),
                               lambda s, b: (s * n_b + b, 0))],
        out_specs=pl.BlockSpec((1, d, _BBLK), lambda s, b: (s, 0, b)),
        out_shape=jax.ShapeDtypeStruct((seq, d, batch), jnp.float32),
    )


def _mask_body(x_ref, o_ref):
    o_ref[...] = (x_ref[...] != 0).astype(jnp.int8)


@functools.lru_cache(maxsize=None)
def _make_mask(seq: int, batch: int):
    return pl.pallas_call(
        _mask_body,
        out_shape=jax.ShapeDtypeStruct((seq, batch), jnp.int8),
    )


def kernel(inputs, emb_table, special_table):
    batch, seq = inputs.shape
    vocab, d = emb_table.shape
    nsp = special_table.shape[0]
    n_valid = vocab - nsp
    n_pairs = ((vocab + 2 * _PBLK - 1) // (2 * _PBLK)) * _PBLK

    emb_t = emb_table.T
    sp8 = jnp.zeros((8, d), special_table.dtype).at[:nsp].set(special_table)
    t128 = _make_combine(vocab, d, n_valid, n_pairs)(emb_t, emb_t, sp8)
    table_lin = t128.reshape(2 * n_pairs, d)

    idx_t = inputs.T
    phys = jnp.where(idx_t < n_pairs, 2 * idx_t, 2 * (idx_t - n_pairs) + 1)
    half = _BBLK // 2
    idx_sc = (phys.reshape(seq, batch // _BBLK, 2, half)
              .transpose(0, 1, 3, 2)
              .reshape(seq * batch // _IPG, _IPG)
              .astype(jnp.int32))

    g = _make_gather(batch * seq, 2 * n_pairs, d)(table_lin, idx_sc)
    g128 = g.reshape(batch * seq // 2, 128)
    out3 = _make_tpose(seq, batch, d)(g128)
    out = out3.transpose(2, 0, 1)

    mask_i8 = _make_mask(seq, batch)(idx_t)
    mask = mask_i8.T.astype(jnp.bool_)
    return out, mask


# speedup vs baseline: 1.2192x; 1.2192x over previous
import functools

import jax
import jax.numpy as jnp
from jax import lax
from jax.experimental import pallas as pl
from jax.experimental.pallas import tpu as pltpu
from jax.experimental.pallas import tpu_sc as plsc

_NC = 2
_NS = 16
_NW = _NC * _NS
_IPG = 128
_CHUNK = 512
_HALF = _CHUNK // 2
_PBLK = 256
_BBLK = 512


@functools.lru_cache(maxsize=None)
def _make_gather(n_rows: int, n_pairs: int, d: int):
    assert n_rows % (_NW * _CHUNK) == 0
    b_per_w = n_rows // _NW
    n_chunks = b_per_w // _CHUNK
    n_sub = _CHUNK // _IPG
    idx_rows_per_w = b_per_w // _IPG
    mesh = plsc.VectorSubcoreMesh(core_axis_name="c", subcore_axis_name="s")

    @functools.partial(
        pl.kernel,
        mesh=mesh,
        out_type=jax.ShapeDtypeStruct((n_rows // 2, 2 * d), jnp.float32),
        scratch_types=[
            pltpu.VMEM((n_sub, _IPG), jnp.int32),
            pltpu.VMEM((_CHUNK, d), jnp.float32),
            pltpu.SemaphoreType.DMA,
        ],
        compiler_params=pltpu.CompilerParams(use_tc_tiling_on_sc=False),
    )
    def gather_kernel(table_hbm, idx_hbm, out_hbm, idx_v, rows_v, sem):
        wid = lax.axis_index("s") * _NC + lax.axis_index("c")
        row_base = wid * b_per_w
        idx_row_base = wid * idx_rows_per_w

        def body(g, carry):
            pltpu.sync_copy(idx_hbm.at[pl.ds(idx_row_base + g * n_sub, n_sub)],
                            idx_v)
            copies = [
                pltpu.async_copy(table_hbm.at[idx_v.at[j]],
                                 rows_v.at[pl.ds(j * _IPG, _IPG)], sem)
                for j in range(n_sub)
            ]
            for c in copies:
                c.wait()
            row0 = (row_base + g * _CHUNK) // 2
            pltpu.sync_copy(rows_v.at[pl.ds(0, _HALF)],
                            out_hbm.at[pl.ds(row0, _HALF), pl.ds(0, d)])
            pltpu.sync_copy(rows_v.at[pl.ds(_HALF, _HALF)],
                            out_hbm.at[pl.ds(row0, _HALF), pl.ds(d, d)])
            return carry

        lax.fori_loop(0, n_chunks, body, 0)

    return gather_kernel


def _combine_body(n_valid, n_pairs, x1_ref, x2_ref, sp_ref, o_ref):
    i = pl.program_id(0)
    halves = []
    for h in range(2):
        a = x1_ref[:, pl.ds(128 * h, 128)].T
        b = x2_ref[:, pl.ds(128 * h, 128)].T
        rows_b = (lax.broadcasted_iota(jnp.int32, (128, 1), 0)
                  + (n_pairs + i * _PBLK + 128 * h))
        s_idx = jnp.clip(rows_b - n_valid, 0, 4)
        for k in range(1, 5):
            b = b + jnp.where(s_idx == k, 1.0, 0.0) * sp_ref[k, :][None, :]
        halves.append(jnp.concatenate([a, b], axis=1))
    o_ref[...] = jnp.concatenate(halves, axis=0)


@functools.lru_cache(maxsize=None)
def _make_combine(vocab: int, d: int, n_valid: int, n_pairs: int):
    n_blk = n_pairs // _PBLK
    return pl.pallas_call(
        functools.partial(_combine_body, n_valid, n_pairs),
        grid=(n_blk,),
        in_specs=[
            pl.BlockSpec((d, _PBLK), lambda i: (0, i)),
            pl.BlockSpec((d, _PBLK), lambda i: (0, n_blk + i)),
            pl.BlockSpec((8, d), lambda i: (0, 0)),
        ],
        out_specs=pl.BlockSpec((_PBLK, 2 * d), lambda i: (i, 0)),
        out_shape=jax.ShapeDtypeStruct((n_pairs, 2 * d), jnp.float32),
    )


def _tpose_body(x_ref, o_ref):
    x = x_ref[...]
    for h in range(2):
        sq = x[128 * h:128 * (h + 1), :].T
        o_ref[0, :, pl.ds(128 * h, 128)] = sq[0:64, :]
        o_ref[0, :, pl.ds(_HALF + 128 * h, 128)] = sq[64:128, :]


@functools.lru_cache(maxsize=None)
def _make_tpose(seq: int, batch: int, d: int):
    n_b = batch // _BBLK
    return pl.pallas_call(
        _tpose_body,
        grid=(seq, n_b),
        in_specs=[pl.BlockSpec((_BBLK // 2, 128),
                               lambda s, b: (s * n_b + b, 0))],
        out_specs=pl.BlockSpec((1, d, _BBLK), lambda s, b: (s, 0, b)),
        out_shape=jax.ShapeDtypeStruct((seq, d, batch), jnp.float32),
    )


def _mask_body(x_ref, o_ref):
    o_ref[...] = (x_ref[...] != 0).astype(jnp.int8)


@functools.lru_cache(maxsize=None)
def _make_mask(seq: int, batch: int):
    return pl.pallas_call(
        _mask_body,
        out_shape=jax.ShapeDtypeStruct((seq, batch), jnp.int8),
    )


def kernel(inputs, emb_table, special_table):
    batch, seq = inputs.shape
    vocab, d = emb_table.shape
    nsp = special_table.shape[0]
    n_valid = vocab - nsp
    n_pairs = ((vocab + 2 * _PBLK - 1) // (2 * _PBLK)) * _PBLK

    emb_t = emb_table.T
    sp8 = jnp.zeros((8, d), special_table.dtype).at[:nsp].set(special_table)
    t128 = _make_combine(vocab, d, n_valid, n_pairs)(emb_t, emb_t, sp8)
    table_lin = t128.reshape(2 * n_pairs, d)

    idx_t = inputs.T
    phys = jnp.where(idx_t < n_pairs, 2 * idx_t, 2 * (idx_t - n_pairs) + 1)
    idx_sc = phys.reshape(seq * batch // _IPG, _IPG).astype(jnp.int32)

    g128 = _make_gather(batch * seq, n_pairs, d)(table_lin, idx_sc)
    out3 = _make_tpose(seq, batch, d)(g128)
    out = out3.transpose(2, 0, 1)

    mask_i8 = _make_mask(seq, batch)(idx_t)
    mask = mask_i8.T.astype(jnp.bool_)
    return out, mask


# speedup vs baseline: 3.1605x; 2.5922x over previous
import functools

import jax
import jax.numpy as jnp
from jax import lax
from jax.experimental import pallas as pl
from jax.experimental.pallas import tpu as pltpu
from jax.experimental.pallas import tpu_sc as plsc

_NC = 2
_NS = 16
_NW = _NC * _NS
_IPG = 128
_CHUNK = 512
_HALF = _CHUNK // 2
_PBLK = 1024
_BBLK = 4096


@functools.lru_cache(maxsize=None)
def _make_gather(n_rows: int, n_pairs: int, d: int):
    assert n_rows % (_NW * _CHUNK) == 0
    b_per_w = n_rows // _NW
    n_chunks = b_per_w // _CHUNK
    n_sub = _CHUNK // _IPG
    idx_rows_per_w = b_per_w // _IPG
    mesh = plsc.VectorSubcoreMesh(core_axis_name="c", subcore_axis_name="s")

    @functools.partial(
        pl.kernel,
        mesh=mesh,
        out_type=jax.ShapeDtypeStruct((n_rows // 2, 2 * d), jnp.float32),
        scratch_types=[
            pltpu.VMEM((n_sub, _IPG), jnp.int32),
            pltpu.VMEM((_CHUNK, d), jnp.float32),
            pltpu.SemaphoreType.DMA,
        ],
        compiler_params=pltpu.CompilerParams(use_tc_tiling_on_sc=False),
    )
    def gather_kernel(table_hbm, idx_hbm, out_hbm, idx_v, rows_v, sem):
        wid = lax.axis_index("s") * _NC + lax.axis_index("c")
        row_base = wid * b_per_w
        idx_row_base = wid * idx_rows_per_w

        def body(g, carry):
            pltpu.sync_copy(idx_hbm.at[pl.ds(idx_row_base + g * n_sub, n_sub)],
                            idx_v)
            copies = [
                pltpu.async_copy(table_hbm.at[idx_v.at[j]],
                                 rows_v.at[pl.ds(j * _IPG, _IPG)], sem)
                for j in range(n_sub)
            ]
            for c in copies:
                c.wait()
            row0 = (row_base + g * _CHUNK) // 2
            pltpu.sync_copy(rows_v.at[pl.ds(0, _HALF)],
                            out_hbm.at[pl.ds(row0, _HALF), pl.ds(0, d)])
            pltpu.sync_copy(rows_v.at[pl.ds(_HALF, _HALF)],
                            out_hbm.at[pl.ds(row0, _HALF), pl.ds(d, d)])
            return carry

        lax.fori_loop(0, n_chunks, body, 0)

    return gather_kernel


def _combine_body(n_valid, n_pairs, x1_ref, x2_ref, sp_ref, o_ref):
    i = pl.program_id(0)
    halves = []
    for h in range(_PBLK // 128):
        a = x1_ref[:, pl.ds(128 * h, 128)].T
        b = x2_ref[:, pl.ds(128 * h, 128)].T
        rows_b = (lax.broadcasted_iota(jnp.int32, (128, 1), 0)
                  + (n_pairs + i * _PBLK + 128 * h))
        s_idx = jnp.clip(rows_b - n_valid, 0, 4)
        for k in range(1, 5):
            b = b + jnp.where(s_idx == k, 1.0, 0.0) * sp_ref[k, :][None, :]
        halves.append(jnp.concatenate([a, b], axis=1))
    o_ref[...] = jnp.concatenate(halves, axis=0)


@functools.lru_cache(maxsize=None)
def _make_combine(vocab: int, d: int, n_valid: int, n_pairs: int):
    n_blk = n_pairs // _PBLK
    return pl.pallas_call(
        functools.partial(_combine_body, n_valid, n_pairs),
        grid=(n_blk,),
        in_specs=[
            pl.BlockSpec((d, _PBLK), lambda i: (0, i)),
            pl.BlockSpec((d, _PBLK), lambda i: (0, n_blk + i)),
            pl.BlockSpec((8, d), lambda i: (0, 0)),
        ],
        out_specs=pl.BlockSpec((_PBLK, 2 * d), lambda i: (i, 0)),
        out_shape=jax.ShapeDtypeStruct((n_pairs, 2 * d), jnp.float32),
    )


def _tpose_body(x_ref, o_ref):
    x = x_ref[...]
    for h in range(_BBLK // 256):
        sq = x[128 * h:128 * (h + 1), :].T
        lane0 = 512 * (h // 2) + 128 * (h % 2)
        o_ref[0, :, pl.ds(lane0, 128)] = sq[0:64, :]
        o_ref[0, :, pl.ds(lane0 + _HALF, 128)] = sq[64:128, :]


@functools.lru_cache(maxsize=None)
def _make_tpose(seq: int, batch: int, d: int):
    n_b = batch // _BBLK
    return pl.pallas_call(
        _tpose_body,
        grid=(seq, n_b),
        in_specs=[pl.BlockSpec((_BBLK // 2, 128),
                               lambda s, b: (s * n_b + b, 0))],
        out_specs=pl.BlockSpec((1, d, _BBLK), lambda s, b: (s, 0, b)),
        out_shape=jax.ShapeDtypeStruct((seq, d, batch), jnp.float32),
    )


def _mask_body(x_ref, o_ref):
    o_ref[...] = (x_ref[...] != 0).astype(jnp.int8)


@functools.lru_cache(maxsize=None)
def _make_mask(seq: int, batch: int):
    return pl.pallas_call(
        _mask_body,
        out_shape=jax.ShapeDtypeStruct((seq, batch), jnp.int8),
    )


def kernel(inputs, emb_table, special_table):
    batch, seq = inputs.shape
    vocab, d = emb_table.shape
    nsp = special_table.shape[0]
    n_valid = vocab - nsp
    n_pairs = ((vocab + 2 * _PBLK - 1) // (2 * _PBLK)) * _PBLK

    emb_t = emb_table.T
    sp8 = jnp.zeros((8, d), special_table.dtype).at[:nsp].set(special_table)
    t128 = _make_combine(vocab, d, n_valid, n_pairs)(emb_t, emb_t, sp8)
    table_lin = t128.reshape(2 * n_pairs, d)

    idx_t = inputs.T
    phys = jnp.where(idx_t < n_pairs, 2 * idx_t, 2 * (idx_t - n_pairs) + 1)
    idx_sc = phys.reshape(seq * batch // _IPG, _IPG).astype(jnp.int32)

    g128 = _make_gather(batch * seq, n_pairs, d)(table_lin, idx_sc)
    out3 = _make_tpose(seq, batch, d)(g128)
    out = out3.transpose(2, 0, 1)

    mask_i8 = _make_mask(seq, batch)(idx_t)
    mask = mask_i8.T.astype(jnp.bool_)
    return out, mask


# speedup vs baseline: 3.4213x; 1.0825x over previous
import functools

import jax
import jax.numpy as jnp
from jax import lax
from jax.experimental import pallas as pl
from jax.experimental.pallas import tpu as pltpu
from jax.experimental.pallas import tpu_sc as plsc

_NC = 2
_NS = 16
_NW = _NC * _NS
_IPG = 128
_CHUNK = 512
_HALF = _CHUNK // 2
_PBLK = 1024
_BBLK = 4096


@functools.lru_cache(maxsize=None)
def _make_gather(n_rows: int, n_pairs: int, d: int):
    assert n_rows % (_NW * _CHUNK) == 0
    b_per_w = n_rows // _NW
    n_chunks = b_per_w // _CHUNK
    n_sub = _CHUNK // _IPG
    idx_rows_per_w = b_per_w // _IPG
    mesh = plsc.VectorSubcoreMesh(core_axis_name="c", subcore_axis_name="s")

    @functools.partial(
        pl.kernel,
        mesh=mesh,
        out_type=jax.ShapeDtypeStruct((n_rows // 2, 2 * d), jnp.float32),
        scratch_types=[
            pltpu.VMEM((2, n_sub, _IPG), jnp.int32),
            pltpu.VMEM((2, _CHUNK, d), jnp.float32),
            pltpu.SemaphoreType.DMA,
            pltpu.SemaphoreType.DMA,
            pltpu.SemaphoreType.DMA,
        ],
        compiler_params=pltpu.CompilerParams(use_tc_tiling_on_sc=False),
    )
    def gather_kernel(table_hbm, idx_hbm, out_hbm, idx_v, rows_v,
                      gsem, osem0, osem1):
        wid = lax.axis_index("s") * _NC + lax.axis_index("c")
        row_base = wid * b_per_w
        idx_row_base = wid * idx_rows_per_w
        osems = (osem0, osem1)

        def step(g, k):
            row0 = (row_base + g * _CHUNK) // 2
            outA = out_hbm.at[pl.ds(row0, _HALF), pl.ds(0, d)]
            outB = out_hbm.at[pl.ds(row0, _HALF), pl.ds(d, d)]

            @pl.when(g >= 2)
            def _drain():
                pltpu.make_async_copy(rows_v.at[k, pl.ds(0, _HALF)],
                                      outA, osems[k]).wait()
                pltpu.make_async_copy(rows_v.at[k, pl.ds(_HALF, _HALF)],
                                      outB, osems[k]).wait()

            pltpu.sync_copy(idx_hbm.at[pl.ds(idx_row_base + g * n_sub, n_sub)],
                            idx_v.at[k])
            copies = [
                pltpu.async_copy(table_hbm.at[idx_v.at[k, j]],
                                 rows_v.at[k, pl.ds(j * _IPG, _IPG)], gsem)
                for j in range(n_sub)
            ]
            for c in copies:
                c.wait()
            pltpu.async_copy(rows_v.at[k, pl.ds(0, _HALF)], outA, osems[k])
            pltpu.async_copy(rows_v.at[k, pl.ds(_HALF, _HALF)], outB,
                             osems[k])

        def body(h, carry):
            step(2 * h, 0)
            step(2 * h + 1, 1)
            return carry

        lax.fori_loop(0, n_chunks // 2, body, 0)
        for k in range(2):
            row0 = (row_base + (n_chunks - 2 + k) * _CHUNK) // 2
            pltpu.make_async_copy(
                rows_v.at[k, pl.ds(0, _HALF)],
                out_hbm.at[pl.ds(row0, _HALF), pl.ds(0, d)], osems[k]).wait()
            pltpu.make_async_copy(
                rows_v.at[k, pl.ds(_HALF, _HALF)],
                out_hbm.at[pl.ds(row0, _HALF), pl.ds(d, d)], osems[k]).wait()

    return gather_kernel


def _combine_body(n_valid, n_pairs, x1_ref, x2_ref, sp_ref, o_ref):
    i = pl.program_id(0)
    halves = []
    for h in range(_PBLK // 128):
        a = x1_ref[:, pl.ds(128 * h, 128)].T
        b = x2_ref[:, pl.ds(128 * h, 128)].T
        rows_b = (lax.broadcasted_iota(jnp.int32, (128, 1), 0)
                  + (n_pairs + i * _PBLK + 128 * h))
        s_idx = jnp.clip(rows_b - n_valid, 0, 4)
        for k in range(1, 5):
            b = b + jnp.where(s_idx == k, 1.0, 0.0) * sp_ref[k, :][None, :]
        halves.append(jnp.concatenate([a, b], axis=1))
    o_ref[...] = jnp.concatenate(halves, axis=0)


@functools.lru_cache(maxsize=None)
def _make_combine(vocab: int, d: int, n_valid: int, n_pairs: int):
    n_blk = n_pairs // _PBLK
    return pl.pallas_call(
        functools.partial(_combine_body, n_valid, n_pairs),
        grid=(n_blk,),
        in_specs=[
            pl.BlockSpec((d, _PBLK), lambda i: (0, i)),
            pl.BlockSpec((d, _PBLK), lambda i: (0, n_blk + i)),
            pl.BlockSpec((8, d), lambda i: (0, 0)),
        ],
        out_specs=pl.BlockSpec((_PBLK, 2 * d), lambda i: (i, 0)),
        out_shape=jax.ShapeDtypeStruct((n_pairs, 2 * d), jnp.float32),
    )


def _tpose_body(x_ref, o_ref):
    x = x_ref[...]
    for h in range(_BBLK // 256):
        sq = x[128 * h:128 * (h + 1), :].T
        lane0 = 512 * (h // 2) + 128 * (h % 2)
        o_ref[0, :, pl.ds(lane0, 128)] = sq[0:64, :]
        o_ref[0, :, pl.ds(lane0 + _HALF, 128)] = sq[64:128, :]


@functools.lru_cache(maxsize=None)
def _make_tpose(seq: int, batch: int, d: int):
    n_b = batch // _BBLK
    return pl.pallas_call(
        _tpose_body,
        grid=(seq, n_b),
        in_specs=[pl.BlockSpec((_BBLK // 2, 128),
                               lambda s, b: (s * n_b + b, 0))],
        out_specs=pl.BlockSpec((1, d, _BBLK), lambda s, b: (s, 0, b)),
        out_shape=jax.ShapeDtypeStruct((seq, d, batch), jnp.float32),
    )


def _mask_body(x_ref, o_ref):
    o_ref[...] = (x_ref[...] != 0).astype(jnp.int8)


@functools.lru_cache(maxsize=None)
def _make_mask(seq: int, batch: int):
    return pl.pallas_call(
        _mask_body,
        out_shape=jax.ShapeDtypeStruct((seq, batch), jnp.int8),
    )


def kernel(inputs, emb_table, special_table):
    batch, seq = inputs.shape
    vocab, d = emb_table.shape
    nsp = special_table.shape[0]
    n_valid = vocab - nsp
    n_pairs = ((vocab + 2 * _PBLK - 1) // (2 * _PBLK)) * _PBLK

    emb_t = emb_table.T
    sp8 = jnp.zeros((8, d), special_table.dtype).at[:nsp].set(special_table)
    t128 = _make_combine(vocab, d, n_valid, n_pairs)(emb_t, emb_t, sp8)
    table_lin = t128.reshape(2 * n_pairs, d)

    idx_t = inputs.T
    phys = jnp.where(idx_t < n_pairs, 2 * idx_t, 2 * (idx_t - n_pairs) + 1)
    idx_sc = phys.reshape(seq * batch // _IPG, _IPG).astype(jnp.int32)

    g128 = _make_gather(batch * seq, n_pairs, d)(table_lin, idx_sc)
    out3 = _make_tpose(seq, batch, d)(g128)
    out = out3.transpose(2, 0, 1)

    mask_i8 = _make_mask(seq, batch)(idx_t)
    mask = mask_i8.T.astype(jnp.bool_)
    return out, mask


# speedup vs baseline: 3.4758x; 1.0159x over previous
import functools

import jax
import jax.numpy as jnp
from jax import lax
from jax.experimental import pallas as pl
from jax.experimental.pallas import tpu as pltpu
from jax.experimental.pallas import tpu_sc as plsc

_NC = 2
_NS = 16
_NW = _NC * _NS
_IPG = 128
_CHUNK = 256
_HALF = _CHUNK // 2
_PBLK = 1024
_BBLK = 4096


@functools.lru_cache(maxsize=None)
def _make_gather(n_rows: int, n_pairs: int, d: int):
    assert n_rows % (_NW * _CHUNK) == 0
    b_per_w = n_rows // _NW
    n_chunks = b_per_w // _CHUNK
    n_sub = _CHUNK // _IPG
    idx_rows_per_w = b_per_w // _IPG
    mesh = plsc.VectorSubcoreMesh(core_axis_name="c", subcore_axis_name="s")

    @functools.partial(
        pl.kernel,
        mesh=mesh,
        out_type=jax.ShapeDtypeStruct((n_rows // 2, 2 * d), jnp.float32),
        scratch_types=[
            pltpu.VMEM((2, n_sub, _IPG), jnp.int32),
            pltpu.VMEM((2, _CHUNK, d), jnp.float32),
            pltpu.SemaphoreType.DMA,
            pltpu.SemaphoreType.DMA,
            pltpu.SemaphoreType.DMA,
        ],
        compiler_params=pltpu.CompilerParams(use_tc_tiling_on_sc=False),
    )
    def gather_kernel(table_hbm, idx_hbm, out_hbm, idx_v, rows_v,
                      gsem, osem0, osem1):
        wid = lax.axis_index("s") * _NC + lax.axis_index("c")
        row_base = wid * b_per_w
        idx_row_base = wid * idx_rows_per_w
        osems = (osem0, osem1)

        def step(g, k):
            row0 = (row_base + g * _CHUNK) // 2
            outA = out_hbm.at[pl.ds(row0, _HALF), pl.ds(0, d)]
            outB = out_hbm.at[pl.ds(row0, _HALF), pl.ds(d, d)]

            @pl.when(g >= 2)
            def _drain():
                pltpu.make_async_copy(rows_v.at[k, pl.ds(0, _HALF)],
                                      outA, osems[k]).wait()
                pltpu.make_async_copy(rows_v.at[k, pl.ds(_HALF, _HALF)],
                                      outB, osems[k]).wait()

            pltpu.sync_copy(idx_hbm.at[pl.ds(idx_row_base + g * n_sub, n_sub)],
                            idx_v.at[k])
            copies = [
                pltpu.async_copy(table_hbm.at[idx_v.at[k, j]],
                                 rows_v.at[k, pl.ds(j * _IPG, _IPG)], gsem)
                for j in range(n_sub)
            ]
            for c in copies:
                c.wait()
            pltpu.async_copy(rows_v.at[k, pl.ds(0, _HALF)], outA, osems[k])
            pltpu.async_copy(rows_v.at[k, pl.ds(_HALF, _HALF)], outB,
                             osems[k])

        def body(h, carry):
            step(2 * h, 0)
            step(2 * h + 1, 1)
            return carry

        lax.fori_loop(0, n_chunks // 2, body, 0)
        for k in range(2):
            row0 = (row_base + (n_chunks - 2 + k) * _CHUNK) // 2
            pltpu.make_async_copy(
                rows_v.at[k, pl.ds(0, _HALF)],
                out_hbm.at[pl.ds(row0, _HALF), pl.ds(0, d)], osems[k]).wait()
            pltpu.make_async_copy(
                rows_v.at[k, pl.ds(_HALF, _HALF)],
                out_hbm.at[pl.ds(row0, _HALF), pl.ds(d, d)], osems[k]).wait()

    return gather_kernel


def _combine_body(n_valid, n_pairs, x1_ref, x2_ref, sp_ref, o_ref):
    i = pl.program_id(0)
    halves = []
    for h in range(_PBLK // 128):
        a = x1_ref[:, pl.ds(128 * h, 128)].T
        b = x2_ref[:, pl.ds(128 * h, 128)].T
        rows_b = (lax.broadcasted_iota(jnp.int32, (128, 1), 0)
                  + (n_pairs + i * _PBLK + 128 * h))
        s_idx = jnp.clip(rows_b - n_valid, 0, 4)
        for k in range(1, 5):
            b = b + jnp.where(s_idx == k, 1.0, 0.0) * sp_ref[k, :][None, :]
        halves.append(jnp.concatenate([a, b], axis=1))
    o_ref[...] = jnp.concatenate(halves, axis=0)


@functools.lru_cache(maxsize=None)
def _make_combine(vocab: int, d: int, n_valid: int, n_pairs: int):
    n_blk = n_pairs // _PBLK
    return pl.pallas_call(
        functools.partial(_combine_body, n_valid, n_pairs),
        grid=(n_blk,),
        in_specs=[
            pl.BlockSpec((d, _PBLK), lambda i: (0, i)),
            pl.BlockSpec((d, _PBLK), lambda i: (0, n_blk + i)),
            pl.BlockSpec((8, d), lambda i: (0, 0)),
        ],
        out_specs=pl.BlockSpec((_PBLK, 2 * d), lambda i: (i, 0)),
        out_shape=jax.ShapeDtypeStruct((n_pairs, 2 * d), jnp.float32),
    )


def _tpose_body(x_ref, o_ref):
    x = x_ref[...]
    for h in range(_BBLK // 256):
        sq = x[128 * h:128 * (h + 1), :].T
        o_ref[0, :, pl.ds(256 * h, 128)] = sq[0:64, :]
        o_ref[0, :, pl.ds(256 * h + _HALF, 128)] = sq[64:128, :]


def _tpose_alias_body(x_ref, prev_ref, o_ref):
    del prev_ref
    _tpose_body(x_ref, o_ref)


@functools.lru_cache(maxsize=None)
def _make_tpose(seq: int, batch: int, d: int, n_seq: int, s_off: int,
                alias: bool):
    n_b = batch // _BBLK
    g_spec = pl.BlockSpec((_BBLK // 2, 128), lambda s, b: (s * n_b + b, 0))
    out_spec = pl.BlockSpec((1, d, _BBLK), lambda s, b: (s + s_off, 0, b))
    out_shape = jax.ShapeDtypeStruct((seq, d, batch), jnp.float32)
    if not alias:
        return pl.pallas_call(_tpose_body, grid=(n_seq, n_b),
                              in_specs=[g_spec], out_specs=out_spec,
                              out_shape=out_shape)
    return pl.pallas_call(
        _tpose_alias_body,
        grid=(n_seq, n_b),
        in_specs=[g_spec, pl.BlockSpec(memory_space=pl.ANY)],
        out_specs=out_spec,
        out_shape=out_shape,
        input_output_aliases={1: 0},
    )


def _mask_body(x_ref, o_ref):
    o_ref[...] = (x_ref[...] != 0).astype(jnp.int8)


@functools.lru_cache(maxsize=None)
def _make_mask(seq: int, batch: int):
    return pl.pallas_call(
        _mask_body,
        out_shape=jax.ShapeDtypeStruct((seq, batch), jnp.int8),
    )


def kernel(inputs, emb_table, special_table):
    batch, seq = inputs.shape
    vocab, d = emb_table.shape
    nsp = special_table.shape[0]
    n_valid = vocab - nsp
    n_pairs = ((vocab + 2 * _PBLK - 1) // (2 * _PBLK)) * _PBLK

    emb_t = emb_table.T
    sp8 = jnp.zeros((8, d), special_table.dtype).at[:nsp].set(special_table)
    t128 = _make_combine(vocab, d, n_valid, n_pairs)(emb_t, emb_t, sp8)
    table_lin = t128.reshape(2 * n_pairs, d)

    idx_t = inputs.T
    phys = jnp.where(idx_t < n_pairs, 2 * idx_t, 2 * (idx_t - n_pairs) + 1)
    idx_sc = phys.reshape(seq * batch // _IPG, _IPG).astype(jnp.int32)

    half_seq = seq // 2
    n_half_rows = (batch * half_seq) // _IPG
    g1 = _make_gather(batch * half_seq, n_pairs, d)(
        table_lin, idx_sc[:n_half_rows])
    g2 = _make_gather(batch * half_seq, n_pairs, d)(
        table_lin, idx_sc[n_half_rows:])
    out3a = _make_tpose(seq, batch, d, half_seq, 0, False)(g1)
    out3 = _make_tpose(seq, batch, d, half_seq, half_seq, True)(g2, out3a)
    out = out3.transpose(2, 0, 1)

    mask_i8 = _make_mask(seq, batch)(idx_t)
    mask = mask_i8.T.astype(jnp.bool_)
    return out, mask


# speedup vs baseline: 3.7835x; 1.0885x over previous
import functools

import jax
import jax.numpy as jnp
from jax import lax
from jax.experimental import pallas as pl
from jax.experimental.pallas import tpu as pltpu
from jax.experimental.pallas import tpu_sc as plsc

_NC = 2
_NS = 16
_NW = _NC * _NS
_IPG = 128
_CHUNK = 256
_HALF = _CHUNK // 2
_PBLK = 1024
_BBLK = 16384


@functools.lru_cache(maxsize=None)
def _make_gather(n_rows: int, n_pairs: int, d: int):
    assert n_rows % (_NW * _CHUNK) == 0
    b_per_w = n_rows // _NW
    n_chunks = b_per_w // _CHUNK
    n_sub = _CHUNK // _IPG
    idx_rows_per_w = b_per_w // _IPG
    mesh = plsc.VectorSubcoreMesh(core_axis_name="c", subcore_axis_name="s")

    @functools.partial(
        pl.kernel,
        mesh=mesh,
        out_type=jax.ShapeDtypeStruct((n_rows // 2, 2 * d), jnp.float32),
        scratch_types=[
            pltpu.VMEM((2, n_sub, _IPG), jnp.int32),
            pltpu.VMEM((2, _CHUNK, d), jnp.float32),
            pltpu.SemaphoreType.DMA,
            pltpu.SemaphoreType.DMA,
            pltpu.SemaphoreType.DMA,
        ],
        compiler_params=pltpu.CompilerParams(use_tc_tiling_on_sc=False),
    )
    def gather_kernel(table_hbm, idx_hbm, out_hbm, idx_v, rows_v,
                      gsem, osem0, osem1):
        wid = lax.axis_index("s") * _NC + lax.axis_index("c")
        row_base = wid * b_per_w
        idx_row_base = wid * idx_rows_per_w
        osems = (osem0, osem1)

        def step(g, k):
            row0 = (row_base + g * _CHUNK) // 2
            outA = out_hbm.at[pl.ds(row0, _HALF), pl.ds(0, d)]
            outB = out_hbm.at[pl.ds(row0, _HALF), pl.ds(d, d)]

            @pl.when(g >= 2)
            def _drain():
                pltpu.make_async_copy(rows_v.at[k, pl.ds(0, _HALF)],
                                      outA, osems[k]).wait()
                pltpu.make_async_copy(rows_v.at[k, pl.ds(_HALF, _HALF)],
                                      outB, osems[k]).wait()

            pltpu.sync_copy(idx_hbm.at[pl.ds(idx_row_base + g * n_sub, n_sub)],
                            idx_v.at[k])
            copies = [
                pltpu.async_copy(table_hbm.at[idx_v.at[k, j]],
                                 rows_v.at[k, pl.ds(j * _IPG, _IPG)], gsem)
                for j in range(n_sub)
            ]
            for c in copies:
                c.wait()
            pltpu.async_copy(rows_v.at[k, pl.ds(0, _HALF)], outA, osems[k])
            pltpu.async_copy(rows_v.at[k, pl.ds(_HALF, _HALF)], outB,
                             osems[k])

        def body(h, carry):
            step(2 * h, 0)
            step(2 * h + 1, 1)
            return carry

        lax.fori_loop(0, n_chunks // 2, body, 0)
        for k in range(2):
            row0 = (row_base + (n_chunks - 2 + k) * _CHUNK) // 2
            pltpu.make_async_copy(
                rows_v.at[k, pl.ds(0, _HALF)],
                out_hbm.at[pl.ds(row0, _HALF), pl.ds(0, d)], osems[k]).wait()
            pltpu.make_async_copy(
                rows_v.at[k, pl.ds(_HALF, _HALF)],
                out_hbm.at[pl.ds(row0, _HALF), pl.ds(d, d)], osems[k]).wait()

    return gather_kernel


def _combine_body(n_valid, n_pairs, x1_ref, x2_ref, sp_ref, o_ref):
    i = pl.program_id(0)
    halves = []
    for h in range(_PBLK // 128):
        a = x1_ref[:, pl.ds(128 * h, 128)].T
        b = x2_ref[:, pl.ds(128 * h, 128)].T
        rows_b = (lax.broadcasted_iota(jnp.int32, (128, 1), 0)
                  + (n_pairs + i * _PBLK + 128 * h))
        s_idx = jnp.clip(rows_b - n_valid, 0, 4)
        for k in range(1, 5):
            b = b + jnp.where(s_idx == k, 1.0, 0.0) * sp_ref[k, :][None, :]
        halves.append(jnp.concatenate([a, b], axis=1))
    o_ref[...] = jnp.concatenate(halves, axis=0)


@functools.lru_cache(maxsize=None)
def _make_combine(vocab: int, d: int, n_valid: int, n_pairs: int):
    n_blk = n_pairs // _PBLK
    return pl.pallas_call(
        functools.partial(_combine_body, n_valid, n_pairs),
        grid=(n_blk,),
        in_specs=[
            pl.BlockSpec((d, _PBLK), lambda i: (0, i)),
            pl.BlockSpec((d, _PBLK), lambda i: (0, n_blk + i)),
            pl.BlockSpec((8, d), lambda i: (0, 0)),
        ],
        out_specs=pl.BlockSpec((_PBLK, 2 * d), lambda i: (i, 0)),
        out_shape=jax.ShapeDtypeStruct((n_pairs, 2 * d), jnp.float32),
    )


def _tpose_body(x_ref, o_ref):
    x = x_ref[...]
    for h in range(_BBLK // 256):
        sq = x[128 * h:128 * (h + 1), :].T
        o_ref[0, :, pl.ds(256 * h, 128)] = sq[0:64, :]
        o_ref[0, :, pl.ds(256 * h + _HALF, 128)] = sq[64:128, :]


def _tpose_alias_body(x_ref, prev_ref, o_ref):
    del prev_ref
    _tpose_body(x_ref, o_ref)


@functools.lru_cache(maxsize=None)
def _make_tpose(seq: int, batch: int, d: int, n_seq: int, s_off: int,
                alias: bool):
    n_b = batch // _BBLK
    g_spec = pl.BlockSpec((_BBLK // 2, 128), lambda s, b: (s * n_b + b, 0))
    out_spec = pl.BlockSpec((1, d, _BBLK), lambda s, b: (s + s_off, 0, b))
    out_shape = jax.ShapeDtypeStruct((seq, d, batch), jnp.float32)
    if not alias:
        return pl.pallas_call(_tpose_body, grid=(n_seq, n_b),
                              in_specs=[g_spec], out_specs=out_spec,
                              out_shape=out_shape)
    return pl.pallas_call(
        _tpose_alias_body,
        grid=(n_seq, n_b),
        in_specs=[g_spec, pl.BlockSpec(memory_space=pl.ANY)],
        out_specs=out_spec,
        out_shape=out_shape,
        input_output_aliases={1: 0},
    )


def _mask_body(x_ref, o_ref):
    o_ref[...] = (x_ref[...] != 0).astype(jnp.int8)


@functools.lru_cache(maxsize=None)
def _make_mask(seq: int, batch: int):
    return pl.pallas_call(
        _mask_body,
        out_shape=jax.ShapeDtypeStruct((seq, batch), jnp.int8),
    )


def kernel(inputs, emb_table, special_table):
    batch, seq = inputs.shape
    vocab, d = emb_table.shape
    nsp = special_table.shape[0]
    n_valid = vocab - nsp
    n_pairs = ((vocab + 2 * _PBLK - 1) // (2 * _PBLK)) * _PBLK

    emb_t = emb_table.T
    sp8 = jnp.zeros((8, d), special_table.dtype).at[:nsp].set(special_table)
    t128 = _make_combine(vocab, d, n_valid, n_pairs)(emb_t, emb_t, sp8)
    table_lin = t128.reshape(2 * n_pairs, d)

    idx_t = inputs.T
    phys = jnp.where(idx_t < n_pairs, 2 * idx_t, 2 * (idx_t - n_pairs) + 1)
    idx_sc = phys.reshape(seq * batch // _IPG, _IPG).astype(jnp.int32)

    half_seq = seq // 2
    n_half_rows = (batch * half_seq) // _IPG
    g1 = _make_gather(batch * half_seq, n_pairs, d)(
        table_lin, idx_sc[:n_half_rows])
    g2 = _make_gather(batch * half_seq, n_pairs, d)(
        table_lin, idx_sc[n_half_rows:])
    out3a = _make_tpose(seq, batch, d, half_seq, 0, False)(g1)
    out3 = _make_tpose(seq, batch, d, half_seq, half_seq, True)(g2, out3a)
    out = out3.transpose(2, 0, 1)

    mask_i8 = _make_mask(seq, batch)(idx_t)
    mask = mask_i8.T.astype(jnp.bool_)
    return out, mask
